# baseline (speedup 1.0000x reference)
import functools
import math

import jax
import jax.numpy as jnp
from jax import lax
from jax.experimental import pallas as pl
from jax.experimental.pallas import tpu as pltpu
from jax.experimental.pallas import tpu_sc as plsc

_VOCAB = 22
_D_OUT = 56
_NC = 2
_NS = 16
_NW = _NC * _NS
_CHUNK = 128


def _table_body(aa_ref, gt_ref, g_ids_ref, props_ref, w1t_ref, b1_ref,
                ln1g_ref, ln1b_ref, w2t_ref, b2_ref, ng_ref, nb_ref, out_ref):
    f32 = jnp.float32
    aa = aa_ref[...]
    gids = g_ids_ref[...]
    onehot = (gids == lax.broadcasted_iota(jnp.int32, (_VOCAB, 5), 1)).astype(f32)
    group_emb = jnp.dot(onehot, gt_ref[...], preferred_element_type=f32)

    h = jnp.dot(props_ref[...], w1t_ref[...], preferred_element_type=f32)
    h = h + b1_ref[...]
    mean = jnp.mean(h, axis=1, keepdims=True)
    var = jnp.mean((h - mean) ** 2, axis=1, keepdims=True)
    h = (h - mean) * lax.rsqrt(var + 1e-5) * ln1g_ref[...] + ln1b_ref[...]
    h = 0.5 * h * (1.0 + lax.erf(h / math.sqrt(2.0)))
    prop_emb = jnp.dot(h, w2t_ref[...], preferred_element_type=f32) + b2_ref[...]

    comb = jnp.concatenate([aa, group_emb, prop_emb], axis=1)
    mean2 = jnp.mean(comb, axis=1, keepdims=True)
    var2 = jnp.mean((comb - mean2) ** 2, axis=1, keepdims=True)
    out_ref[...] = ((comb - mean2) * lax.rsqrt(var2 + 1e-5) * ng_ref[...]
                    + nb_ref[...])


def _build_table(aa_table, group_table, aa_to_group, aa_properties,
                 W1, b1, ln1_g, ln1_b, W2, b2, norm_g, norm_b):
    return pl.pallas_call(
        _table_body,
        out_shape=jax.ShapeDtypeStruct((_VOCAB, _D_OUT), jnp.float32),
    )(aa_table, group_table, aa_to_group.reshape(_VOCAB, 1).astype(jnp.int32),
      aa_properties, W1.T, b1.reshape(1, -1), ln1_g.reshape(1, -1),
      ln1_b.reshape(1, -1), W2.T, b2.reshape(1, -1), norm_g.reshape(1, -1),
      norm_b.reshape(1, -1))


_G = 4
_GROWS = _G * _CHUNK


def _gather_body(table_hbm, idx_hbm, out_hbm, idx_v, st0, st1,
                 gsem0, gsem1, wsem0, wsem1):
    n_chunks_w = idx_v.shape[0]
    ng = n_chunks_w // _G
    wid = lax.axis_index("s") * _NC + lax.axis_index("c")
    cb = wid * n_chunks_w
    ob = wid * n_chunks_w * _CHUNK
    pltpu.sync_copy(idx_hbm.at[pl.ds(cb, n_chunks_w)], idx_v)

    def start_gathers(g, st, gsem):
        for b in range(_G):
            pltpu.async_copy(table_hbm.at[idx_v.at[g * _G + b]],
                             st.at[pl.ds(b * _CHUNK, _CHUNK)], gsem)

    def drain_gathers(st, gsem):
        for b in range(_G):
            pltpu.make_async_copy(table_hbm.at[idx_v.at[0]],
                                  st.at[pl.ds(b * _CHUNK, _CHUNK)], gsem).wait()

    def start_write(g, st, wsem):
        pltpu.async_copy(st, out_hbm.at[pl.ds(ob + g * _GROWS, _GROWS)], wsem)

    def wait_write(st, wsem):
        pltpu.make_async_copy(out_hbm.at[pl.ds(0, _GROWS)], st, wsem).wait()

    start_gathers(0, st0, gsem0)

    def outer(go, carry):
        g0 = go * 2
        g1 = g0 + 1
        @pl.when(go >= 1)
        def _():
            wait_write(st1, wsem1)
        start_gathers(g1, st1, gsem1)
        drain_gathers(st0, gsem0)
        start_write(g0, st0, wsem0)
        wait_write(st0, wsem0)
        @pl.when(go < ng // 2 - 1)
        def _():
            start_gathers(g1 + 1, st0, gsem0)
        drain_gathers(st1, gsem1)
        start_write(g1, st1, wsem1)
        return carry

    lax.fori_loop(0, ng // 2, outer, 0)
    wait_write(st1, wsem1)


def _gather(table, idx_2d, n_tokens):
    n_chunks_w = idx_2d.shape[0] // _NW
    mesh = plsc.VectorSubcoreMesh(core_axis_name="c", subcore_axis_name="s")
    return pl.kernel(
        _gather_body,
        out_type=jax.ShapeDtypeStruct((n_tokens, _D_OUT), jnp.float32),
        mesh=mesh,
        scratch_types=[
            pltpu.VMEM((n_chunks_w, _CHUNK), jnp.int32),
            pltpu.VMEM((_GROWS, _D_OUT), jnp.float32),
            pltpu.VMEM((_GROWS, _D_OUT), jnp.float32),
            pltpu.SemaphoreType.DMA,
            pltpu.SemaphoreType.DMA,
            pltpu.SemaphoreType.DMA,
            pltpu.SemaphoreType.DMA,
        ],
        compiler_params=pltpu.CompilerParams(use_tc_tiling_on_sc=False),
    )(table, idx_2d)


def kernel(token_indices, aa_table, group_table, aa_to_group, aa_properties,
           W1, b1, ln1_g, ln1_b, W2, b2, norm_g, norm_b):
    n_rows, n_cols = token_indices.shape
    n_tokens = n_rows * n_cols
    table = _build_table(aa_table, group_table, aa_to_group, aa_properties,
                         W1, b1, ln1_g, ln1_b, W2, b2, norm_g, norm_b)
    idx_2d = token_indices.reshape(n_tokens // _CHUNK, _CHUNK).astype(jnp.int32)
    out = _gather(table, idx_2d, n_tokens)
    return out.reshape(n_rows, n_cols, _D_OUT)


# speedup vs baseline: 3.5771x; 3.5771x over previous
import functools
import math

import jax
import jax.numpy as jnp
from jax import lax
from jax.experimental import pallas as pl
from jax.experimental.pallas import tpu as pltpu
from jax.experimental.pallas import tpu_sc as plsc

_VOCAB = 22
_D_OUT = 56
_NC = 2
_NS = 16
_NW = _NC * _NS
_CHUNK = 128


def _table_body(aa_ref, gt_ref, g_ids_ref, props_ref, w1t_ref, b1_ref,
                ln1g_ref, ln1b_ref, w2t_ref, b2_ref, ng_ref, nb_ref, out_ref):
    f32 = jnp.float32
    aa = aa_ref[...]
    gids = g_ids_ref[...]
    onehot = (gids == lax.broadcasted_iota(jnp.int32, (_VOCAB, 5), 1)).astype(f32)
    group_emb = jnp.dot(onehot, gt_ref[...], preferred_element_type=f32)

    h = jnp.dot(props_ref[...], w1t_ref[...], preferred_element_type=f32)
    h = h + b1_ref[...]
    mean = jnp.mean(h, axis=1, keepdims=True)
    var = jnp.mean((h - mean) ** 2, axis=1, keepdims=True)
    h = (h - mean) * lax.rsqrt(var + 1e-5) * ln1g_ref[...] + ln1b_ref[...]
    h = 0.5 * h * (1.0 + lax.erf(h / math.sqrt(2.0)))
    prop_emb = jnp.dot(h, w2t_ref[...], preferred_element_type=f32) + b2_ref[...]

    comb = jnp.concatenate([aa, group_emb, prop_emb], axis=1)
    mean2 = jnp.mean(comb, axis=1, keepdims=True)
    var2 = jnp.mean((comb - mean2) ** 2, axis=1, keepdims=True)
    out_ref[...] = ((comb - mean2) * lax.rsqrt(var2 + 1e-5) * ng_ref[...]
                    + nb_ref[...])


def _build_table(aa_table, group_table, aa_to_group, aa_properties,
                 W1, b1, ln1_g, ln1_b, W2, b2, norm_g, norm_b):
    return pl.pallas_call(
        _table_body,
        out_shape=jax.ShapeDtypeStruct((_VOCAB, _D_OUT), jnp.float32),
    )(aa_table, group_table, aa_to_group.reshape(_VOCAB, 1).astype(jnp.int32),
      aa_properties, W1.T, b1.reshape(1, -1), ln1_g.reshape(1, -1),
      ln1_b.reshape(1, -1), W2.T, b2.reshape(1, -1), norm_g.reshape(1, -1),
      norm_b.reshape(1, -1))


_G = 4
_GROWS = _G * _CHUNK


def _gather_body(table_hbm, idx_hbm, out_hbm, table_sh, idx_v, st0, st1,
                 gsem0, gsem1, wsem0, wsem1):
    n_chunks_w = idx_v.shape[0]
    ng = n_chunks_w // _G
    wid = lax.axis_index("s") * _NC + lax.axis_index("c")
    cb = wid * n_chunks_w
    ob = wid * n_chunks_w * _CHUNK

    @pl.when(lax.axis_index("s") == 0)
    def _():
        pltpu.sync_copy(table_hbm, table_sh)
    plsc.subcore_barrier()

    pltpu.sync_copy(idx_hbm.at[pl.ds(cb, n_chunks_w)], idx_v)

    def start_gathers(g, st, gsem):
        for b in range(_G):
            pltpu.async_copy(table_sh.at[idx_v.at[g * _G + b]],
                             st.at[pl.ds(b * _CHUNK, _CHUNK)], gsem)

    def drain_gathers(st, gsem):
        for b in range(_G):
            pltpu.make_async_copy(table_sh.at[idx_v.at[0]],
                                  st.at[pl.ds(b * _CHUNK, _CHUNK)], gsem).wait()

    def start_write(g, st, wsem):
        pltpu.async_copy(st, out_hbm.at[pl.ds(ob + g * _GROWS, _GROWS)], wsem)

    def wait_write(st, wsem):
        pltpu.make_async_copy(out_hbm.at[pl.ds(0, _GROWS)], st, wsem).wait()

    start_gathers(0, st0, gsem0)

    def outer(go, carry):
        g0 = go * 2
        g1 = g0 + 1
        @pl.when(go >= 1)
        def _():
            wait_write(st1, wsem1)
        start_gathers(g1, st1, gsem1)
        drain_gathers(st0, gsem0)
        start_write(g0, st0, wsem0)
        wait_write(st0, wsem0)
        @pl.when(go < ng // 2 - 1)
        def _():
            start_gathers(g1 + 1, st0, gsem0)
        drain_gathers(st1, gsem1)
        start_write(g1, st1, wsem1)
        return carry

    lax.fori_loop(0, ng // 2, outer, 0)
    wait_write(st1, wsem1)


def _gather(table, idx_2d, n_tokens):
    n_chunks_w = idx_2d.shape[0] // _NW
    mesh = plsc.VectorSubcoreMesh(core_axis_name="c", subcore_axis_name="s")
    return pl.kernel(
        _gather_body,
        out_type=jax.ShapeDtypeStruct((n_tokens, _D_OUT), jnp.float32),
        mesh=mesh,
        scratch_types=[
            pltpu.VMEM_SHARED((_VOCAB, _D_OUT), jnp.float32),
            pltpu.VMEM((n_chunks_w, _CHUNK), jnp.int32),
            pltpu.VMEM((_GROWS, _D_OUT), jnp.float32),
            pltpu.VMEM((_GROWS, _D_OUT), jnp.float32),
            pltpu.SemaphoreType.DMA,
            pltpu.SemaphoreType.DMA,
            pltpu.SemaphoreType.DMA,
            pltpu.SemaphoreType.DMA,
        ],
        compiler_params=pltpu.CompilerParams(use_tc_tiling_on_sc=False),
    )(table, idx_2d)


def kernel(token_indices, aa_table, group_table, aa_to_group, aa_properties,
           W1, b1, ln1_g, ln1_b, W2, b2, norm_g, norm_b):
    n_rows, n_cols = token_indices.shape
    n_tokens = n_rows * n_cols
    table = _build_table(aa_table, group_table, aa_to_group, aa_properties,
                         W1, b1, ln1_g, ln1_b, W2, b2, norm_g, norm_b)
    idx_2d = token_indices.reshape(n_tokens // _CHUNK, _CHUNK).astype(jnp.int32)
    out = _gather(table, idx_2d, n_tokens)
    return out.reshape(n_rows, n_cols, _D_OUT)


# speedup vs baseline: 6.9281x; 1.9368x over previous
import functools
import math

import jax
import jax.numpy as jnp
from jax import lax
from jax.experimental import pallas as pl
from jax.experimental.pallas import tpu as pltpu
from jax.experimental.pallas import tpu_sc as plsc

_VOCAB = 22
_D_OUT = 56
_NC = 2
_NS = 16
_NW = _NC * _NS
_LANE = 128
_RGRPS = 8
_CGRPS = _NW // _RGRPS
_DT = _D_OUT // 8


def _table_body(aa_ref, gt_ref, g_ids_ref, props_ref, w1t_ref, b1_ref,
                ln1g_ref, ln1b_ref, w2t_ref, b2_ref, ng_ref, nb_ref, out_ref):
    f32 = jnp.float32
    aa = aa_ref[...]
    gids = g_ids_ref[...]
    onehot = (gids == lax.broadcasted_iota(jnp.int32, (_VOCAB, 5), 1)).astype(f32)
    group_emb = jnp.dot(onehot, gt_ref[...], preferred_element_type=f32)

    h = jnp.dot(props_ref[...], w1t_ref[...], preferred_element_type=f32)
    h = h + b1_ref[...]
    mean = jnp.mean(h, axis=1, keepdims=True)
    var = jnp.mean((h - mean) ** 2, axis=1, keepdims=True)
    h = (h - mean) * lax.rsqrt(var + 1e-5) * ln1g_ref[...] + ln1b_ref[...]
    h = 0.5 * h * (1.0 + lax.erf(h / math.sqrt(2.0)))
    prop_emb = jnp.dot(h, w2t_ref[...], preferred_element_type=f32) + b2_ref[...]

    comb = jnp.concatenate([aa, group_emb, prop_emb], axis=1)
    mean2 = jnp.mean(comb, axis=1, keepdims=True)
    var2 = jnp.mean((comb - mean2) ** 2, axis=1, keepdims=True)
    fused = ((comb - mean2) * lax.rsqrt(var2 + 1e-5) * ng_ref[...]
             + nb_ref[...])

    eye = (lax.broadcasted_iota(jnp.int32, (_D_OUT, _D_OUT), 0)
           == lax.broadcasted_iota(jnp.int32, (_D_OUT, _D_OUT), 1)).astype(f32)
    fused_t = lax.dot_general(eye, fused, (((1,), (1,)), ((), ())),
                              precision=lax.Precision.HIGHEST,
                              preferred_element_type=f32)
    place = (lax.broadcasted_iota(jnp.int32, (_VOCAB, _LANE), 0)
             == lax.broadcasted_iota(jnp.int32, (_VOCAB, _LANE), 1)).astype(f32)
    out_ref[...] = jnp.dot(fused_t, place, precision=lax.Precision.HIGHEST,
                           preferred_element_type=f32)


def _build_table(aa_table, group_table, aa_to_group, aa_properties,
                 W1, b1, ln1_g, ln1_b, W2, b2, norm_g, norm_b):
    return pl.pallas_call(
        _table_body,
        out_shape=jax.ShapeDtypeStruct((_D_OUT, _LANE), jnp.float32),
    )(aa_table, group_table, aa_to_group.reshape(_VOCAB, 1).astype(jnp.int32),
      aa_properties, W1.T, b1.reshape(1, -1), ln1_g.reshape(1, -1),
      ln1_b.reshape(1, -1), W2.T, b2.reshape(1, -1), norm_g.reshape(1, -1),
      norm_b.reshape(1, -1))


def _gather_body(table_hbm, idx_hbm, out_hbm, table_v, idx_v, st0, st1,
                 wsem0, wsem1):
    n_cols = idx_hbm.shape[0]
    n_rt = idx_hbm.shape[1]
    cols_w = n_cols // _CGRPS
    rt_w = n_rt // _RGRPS
    wid = lax.axis_index("s") * _NC + lax.axis_index("c")
    rgrp = wid // _CGRPS
    cgrp = wid % _CGRPS
    c0 = cgrp * cols_w
    rt0 = rgrp * rt_w

    pltpu.sync_copy(table_hbm, table_v)
    pltpu.sync_copy(idx_hbm.at[pl.ds(c0, cols_w), pl.ds(rt0, rt_w)], idx_v)

    def compute(cl, st):
        def rt_loop(rtl, carry):
            def grp_loop(g, carry2):
                ids = idx_v[cl, rtl, pl.ds(g * 16, 16)]
                rt8 = rtl * 8
                for d in range(_D_OUT):
                    dvec = jnp.full((16,), d, jnp.int32)
                    val = plsc.load_gather(table_v, [dvec, ids])
                    st[d // 8, rt8 + (d % 8), pl.ds(g * 16, 16)] = val
                return carry2
            return lax.fori_loop(0, 8, grp_loop, carry)
        lax.fori_loop(0, rt_w, rt_loop, 0)

    def start_writes(cl, st, wsem):
        c = c0 + cl
        for dt in range(_DT):
            row = (c * _DT + dt) * (n_rt * 8) + rt0 * 8
            pltpu.async_copy(st.at[dt], out_hbm.at[pl.ds(row, rt_w * 8)], wsem)

    def drain_writes(st, wsem):
        for dt in range(_DT):
            pltpu.make_async_copy(out_hbm.at[pl.ds(0, rt_w * 8)],
                                  st.at[dt], wsem).wait()

    def outer(cp, carry):
        cl0 = cp * 2

        @pl.when(cp >= 1)
        def _():
            drain_writes(st0, wsem0)
        compute(cl0, st0)
        start_writes(cl0, st0, wsem0)

        @pl.when(cp >= 1)
        def _():
            drain_writes(st1, wsem1)
        compute(cl0 + 1, st1)
        start_writes(cl0 + 1, st1, wsem1)
        return carry

    lax.fori_loop(0, cols_w // 2, outer, 0)
    drain_writes(st0, wsem0)
    drain_writes(st1, wsem1)


def _gather(table, idx3, n_out_rows):
    n_cols, n_rt, _ = idx3.shape
    cols_w = n_cols // _CGRPS
    rt_w = n_rt // _RGRPS
    mesh = plsc.VectorSubcoreMesh(core_axis_name="c", subcore_axis_name="s")
    return pl.kernel(
        _gather_body,
        out_type=jax.ShapeDtypeStruct((n_out_rows, _LANE), jnp.float32),
        mesh=mesh,
        scratch_types=[
            pltpu.VMEM((_D_OUT, _LANE), jnp.float32),
            pltpu.VMEM((cols_w, rt_w, _LANE), jnp.int32),
            pltpu.VMEM((_DT, rt_w * 8, _LANE), jnp.float32),
            pltpu.VMEM((_DT, rt_w * 8, _LANE), jnp.float32),
            pltpu.SemaphoreType.DMA,
            pltpu.SemaphoreType.DMA,
        ],
        compiler_params=pltpu.CompilerParams(use_tc_tiling_on_sc=False,
                                             needs_layout_passes=False),
    )(table, idx3)


def kernel(token_indices, aa_table, group_table, aa_to_group, aa_properties,
           W1, b1, ln1_g, ln1_b, W2, b2, norm_g, norm_b):
    n_rows, n_cols = token_indices.shape
    n_rt = n_rows // _LANE
    table = _build_table(aa_table, group_table, aa_to_group, aa_properties,
                         W1, b1, ln1_g, ln1_b, W2, b2, norm_g, norm_b)
    idx3 = token_indices.T.reshape(n_cols, n_rt, _LANE).astype(jnp.int32)
    n_out_rows = n_cols * _DT * n_rt * 8
    out2 = _gather(table, idx3, n_out_rows)
    out5 = out2.reshape(n_cols, _DT, n_rt, 8, _LANE)
    return out5.transpose(2, 4, 0, 1, 3).reshape(n_rows, n_cols, _D_OUT)
